# Initial kernel scaffold; baseline (speedup 1.0000x reference)
import jax
import jax.numpy as jnp
from jax.experimental import pallas as pl


def kernel(x, hd_vecs):
    raise NotImplementedError("write your pallas kernel here")


# speedup vs baseline: 2.5967x; 2.5967x over previous
import jax
import jax.numpy as jnp
from jax.experimental import pallas as pl
from jax.experimental.pallas import tpu as pltpu

_D = 512
_BM = 1024


def _build_b_kernel(v_ref, b_ref):
    V = v_ref[0]
    norm = jnp.sqrt(jnp.sum(V * V, axis=1, keepdims=True)) + 1e-6
    Vn = V / norm

    G = jax.lax.dot_general(Vn, Vn, (((1,), (1,)), ((), ())),
                            preferred_element_type=jnp.float32, precision=jax.lax.Precision.HIGHEST)

    row = jax.lax.broadcasted_iota(jnp.int32, (_D, _D), 0)
    col = jax.lax.broadcasted_iota(jnp.int32, (_D, _D), 1)

    T = jnp.where(row == col, 2.0, 0.0) + jnp.where(
        (row % 2 == 0) & (col == row + 1), -4.0 * G, 0.0)

    m = 2
    while m < _D:
        mask = ((row // (2 * m) == col // (2 * m))
                & (row % (2 * m) < m) & (col % (2 * m) >= m))
        A = jnp.dot(jnp.dot(T, G, preferred_element_type=jnp.float32, precision=jax.lax.Precision.HIGHEST), T,
                    preferred_element_type=jnp.float32, precision=jax.lax.Precision.HIGHEST)
        T = T - jnp.where(mask, A, 0.0)
        m *= 2

    C = jnp.dot(T, Vn, preferred_element_type=jnp.float32, precision=jax.lax.Precision.HIGHEST)
    CtVn = jax.lax.dot_general(C, Vn, (((0,), (0,)), ((), ())),
                               preferred_element_type=jnp.float32, precision=jax.lax.Precision.HIGHEST)
    b_ref[...] = jnp.where(row == col, 1.0, 0.0) - CtVn


def _apply_kernel(x_ref, b_ref, o_ref):
    o_ref[...] = jnp.dot(x_ref[...], b_ref[...],
                         preferred_element_type=jnp.float32)


def kernel(x, hd_vecs):
    n, d = x.shape
    assert d == _D

    B = pl.pallas_call(
        _build_b_kernel,
        out_shape=jax.ShapeDtypeStruct((_D, _D), jnp.float32),
        in_specs=[pl.BlockSpec((1, _D, _D), lambda: (0, 0, 0))],
        out_specs=pl.BlockSpec((_D, _D), lambda: (0, 0)),
    )(hd_vecs)

    out = pl.pallas_call(
        _apply_kernel,
        out_shape=jax.ShapeDtypeStruct((n, d), x.dtype),
        grid=(n // _BM,),
        in_specs=[
            pl.BlockSpec((_BM, d), lambda i: (i, 0)),
            pl.BlockSpec((_D, _D), lambda i: (0, 0)),
        ],
        out_specs=pl.BlockSpec((_BM, d), lambda i: (i, 0)),
        compiler_params=pltpu.CompilerParams(
            dimension_semantics=("parallel",)),
    )(x, B)
    return out


# speedup vs baseline: 2.7503x; 1.0592x over previous
import jax
import jax.numpy as jnp
from jax.experimental import pallas as pl
from jax.experimental.pallas import tpu as pltpu

_D = 512
_CH = 128
_BM = 1024

_HI = jax.lax.Precision.HIGHEST


def _dot(a, b, prec=_HI):
    return jnp.dot(a, b, preferred_element_type=jnp.float32, precision=prec)


def _dot_ta(a, b, prec=_HI):
    return jax.lax.dot_general(a, b, (((0,), (0,)), ((), ())),
                               preferred_element_type=jnp.float32,
                               precision=prec)


def _dot_tb(a, b, prec=_HI):
    return jax.lax.dot_general(a, b, (((1,), (1,)), ((), ())),
                               preferred_element_type=jnp.float32,
                               precision=prec)


def _chunk_t(G, row, col):
    T = jnp.where(row == col, 2.0, 0.0) + jnp.where(
        (row % 2 == 0) & (col == row + 1), -4.0 * G, 0.0)
    m = 2
    while m < _CH:
        mask = ((row // (2 * m) == col // (2 * m))
                & (row % (2 * m) < m) & (col % (2 * m) >= m))
        A = _dot(_dot(T, G), T)
        T = T - jnp.where(mask, A, 0.0)
        m *= 2
    return T


def _build_b_kernel(v_ref, b_ref):
    V = v_ref[0]
    norm = jnp.sqrt(jnp.sum(V * V, axis=1, keepdims=True)) + 1e-6
    Vn = V / norm

    row = jax.lax.broadcasted_iota(jnp.int32, (_CH, _CH), 0)
    col = jax.lax.broadcasted_iota(jnp.int32, (_CH, _CH), 1)

    Ws, Ts = [], []
    for k in range(_D // _CH):
        Wk = Vn[k * _CH:(k + 1) * _CH, :]
        Gk = _dot_tb(Wk, Wk)
        Ws.append(Wk)
        Ts.append(_chunk_t(Gk, row, col))

    Z = _dot_ta(Ts[0], Ws[0])
    rowd = jax.lax.broadcasted_iota(jnp.int32, (_D, _D), 0)
    cold = jax.lax.broadcasted_iota(jnp.int32, (_D, _D), 1)
    P = jnp.where(rowd == cold, 1.0, 0.0) - _dot_ta(Ws[0], Z)
    for k in range(1, _D // _CH):
        WkP = _dot(Ws[k], P)
        Z = _dot_ta(Ts[k], WkP)
        P = P - _dot_ta(Ws[k], Z)
    b_ref[...] = P


def _apply_kernel(x_ref, b_ref, o_ref):
    o_ref[...] = jnp.dot(x_ref[...], b_ref[...],
                         preferred_element_type=jnp.float32)


def kernel(x, hd_vecs):
    n, d = x.shape
    assert d == _D

    B = pl.pallas_call(
        _build_b_kernel,
        out_shape=jax.ShapeDtypeStruct((_D, _D), jnp.float32),
        in_specs=[pl.BlockSpec((1, _D, _D), lambda: (0, 0, 0))],
        out_specs=pl.BlockSpec((_D, _D), lambda: (0, 0)),
    )(hd_vecs)

    out = pl.pallas_call(
        _apply_kernel,
        out_shape=jax.ShapeDtypeStruct((n, d), x.dtype),
        grid=(n // _BM,),
        in_specs=[
            pl.BlockSpec((_BM, d), lambda i: (i, 0)),
            pl.BlockSpec((_D, _D), lambda i: (0, 0)),
        ],
        out_specs=pl.BlockSpec((_BM, d), lambda i: (i, 0)),
        compiler_params=pltpu.CompilerParams(
            dimension_semantics=("parallel",)),
    )(x, B)
    return out


# speedup vs baseline: 3.4318x; 1.2478x over previous
import jax
import jax.numpy as jnp
from jax.experimental import pallas as pl
from jax.experimental.pallas import tpu as pltpu

_D = 512
_CH = 128
_BM = 4096

_HI = jax.lax.Precision.HIGHEST


def _dot(a, b, prec=_HI):
    return jnp.dot(a, b, preferred_element_type=jnp.float32, precision=prec)


def _dot_ta(a, b, prec=_HI):
    return jax.lax.dot_general(a, b, (((0,), (0,)), ((), ())),
                               preferred_element_type=jnp.float32,
                               precision=prec)


def _dot_tb(a, b, prec=_HI):
    return jax.lax.dot_general(a, b, (((1,), (1,)), ((), ())),
                               preferred_element_type=jnp.float32,
                               precision=prec)


def _chunk_t(G, row, col):
    T = jnp.where(row == col, 2.0, 0.0) + jnp.where(
        (row % 2 == 0) & (col == row + 1), -4.0 * G, 0.0)
    m = 2
    while m < _CH:
        mask = ((row // (2 * m) == col // (2 * m))
                & (row % (2 * m) < m) & (col % (2 * m) >= m))
        A = _dot(_dot(T, G), T)
        T = T - jnp.where(mask, A, 0.0)
        m *= 2
    return T


def _build_b_kernel(v_ref, b_ref):
    V = v_ref[0]
    norm = jnp.sqrt(jnp.sum(V * V, axis=1, keepdims=True)) + 1e-6
    Vn = V / norm

    row = jax.lax.broadcasted_iota(jnp.int32, (_CH, _CH), 0)
    col = jax.lax.broadcasted_iota(jnp.int32, (_CH, _CH), 1)

    Ws, Ts = [], []
    for k in range(_D // _CH):
        Wk = Vn[k * _CH:(k + 1) * _CH, :]
        Gk = _dot_tb(Wk, Wk)
        Ws.append(Wk)
        Ts.append(_chunk_t(Gk, row, col))

    Z = _dot_ta(Ts[0], Ws[0])
    rowd = jax.lax.broadcasted_iota(jnp.int32, (_D, _D), 0)
    cold = jax.lax.broadcasted_iota(jnp.int32, (_D, _D), 1)
    P = jnp.where(rowd == cold, 1.0, 0.0) - _dot_ta(Ws[0], Z)
    for k in range(1, _D // _CH):
        WkP = _dot(Ws[k], P)
        Z = _dot_ta(Ts[k], WkP)
        P = P - _dot_ta(Ws[k], Z)
    b_ref[...] = P


def _apply_kernel(x_ref, b_ref, o_ref):
    o_ref[...] = jnp.dot(x_ref[...], b_ref[...],
                         preferred_element_type=jnp.float32)


def kernel(x, hd_vecs):
    n, d = x.shape
    assert d == _D

    B = pl.pallas_call(
        _build_b_kernel,
        out_shape=jax.ShapeDtypeStruct((_D, _D), jnp.float32),
        in_specs=[pl.BlockSpec((1, _D, _D), lambda: (0, 0, 0))],
        out_specs=pl.BlockSpec((_D, _D), lambda: (0, 0)),
    )(hd_vecs)

    out = pl.pallas_call(
        _apply_kernel,
        out_shape=jax.ShapeDtypeStruct((n, d), x.dtype),
        grid=(n // _BM,),
        in_specs=[
            pl.BlockSpec((_BM, d), lambda i: (i, 0)),
            pl.BlockSpec((_D, _D), lambda i: (0, 0)),
        ],
        out_specs=pl.BlockSpec((_BM, d), lambda i: (i, 0)),
        compiler_params=pltpu.CompilerParams(
            dimension_semantics=("parallel",)),
    )(x, B)
    return out
